# baseline (speedup 1.0000x reference)
import jax
import jax.numpy as jnp
from jax import lax
from jax.experimental import pallas as pl
from jax.experimental.pallas import tpu as pltpu
from jax.experimental.pallas import tpu_sc as plsc

B = 4
N = 16384
K = 2048
NTILE = 8
SHARD = N // NTILE
CH = SHARD // 16
UNROLL = 2
BIGF = 3.0e38


def _fps_body(pts_hbm, out_hbm, pts_v, xs, ys, zs, dd, cand_v, candblk,
              outbuf, packed, shared):
    c = lax.axis_index("c")
    s = lax.axis_index("s")
    g = s // NTILE
    r = s % NTILE
    b = c * 2 + g
    base = r * SHARD
    is_r0 = r == 0

    lanes = jnp.arange(16, dtype=jnp.int32)
    zeros16 = jnp.zeros(16, jnp.int32)
    ones16 = jnp.ones(16, jnp.int32)
    twos16 = jnp.full(16, 2, jnp.int32)
    gmask = (lanes // NTILE) == g
    coord_sel = jnp.clip(lanes - 2, 0, 2)
    lanes_k = [k * 16 + lanes for k in range(UNROLL)]

    pltpu.sync_copy(pts_hbm.at[b, pl.ds(base, SHARD), :], pts_v)

    def sep(j, _):
        rows = j * 16 + lanes
        sl = pl.ds(j * 16, 16)
        xs[sl] = plsc.load_gather(pts_v, [rows, zeros16])
        ys[sl] = plsc.load_gather(pts_v, [rows, ones16])
        zs[sl] = plsc.load_gather(pts_v, [rows, twos16])
        dd[sl] = jnp.full((16,), 1e10, jnp.float32)
        return 0
    lax.fori_loop(0, CH, sep, 0)

    def publish(val, gif, lidx, p):
        lidxv = zeros16 + lidx
        coords = plsc.load_gather(pts_v, [lidxv, coord_sel])
        cand = jnp.where(lanes == 0, val,
                         jnp.where(lanes == 1, gif, coords))
        cand_v[...] = cand
        pltpu.sync_copy(cand_v, shared.at[p, s])
        plsc.subcore_barrier()

    def merge(p):
        pltpu.sync_copy(shared.at[p, pl.ds(g * NTILE, NTILE)], candblk)
        row_sel = lanes & (NTILE - 1)
        valid = lanes < NTILE
        vals = plsc.load_gather(candblk, [row_sel, zeros16])
        idxf = plsc.load_gather(candblk, [row_sel, ones16])
        maxv = jnp.max(vals)
        tie = valid & (vals == maxv)
        minidx = jnp.min(jnp.where(tie, idxf, BIGF))
        rowv = plsc.all_reduce_ffs(tie & (idxf == minidx))
        return plsc.load_gather(candblk, [rowv, lanes])

    publish(jnp.where(is_r0, 1.0, 0.0).astype(jnp.float32),
            jnp.where(is_r0, 0.0, BIGF).astype(jnp.float32),
            jnp.int32(0), 0)
    wrow0 = merge(0)
    outbuf[pl.ds(0, 16)] = wrow0

    def step(i, carry):
        lx, ly, lz = carry

        rm0 = jnp.full((16,), -1.0, jnp.float32)

        @plsc.parallel_loop(0, CH, unroll=UNROLL, carry=(rm0, zeros16))
        def upd(j, mcarry):
            rm, ri = mcarry
            sl = pl.ds(j * 16, 16)
            dx = xs[sl] - lx
            dy = ys[sl] - ly
            dz = zs[sl] - lz
            d = (dx * dx + dy * dy) + dz * dz
            dmin = jnp.minimum(dd[sl], d)
            dd[sl] = dmin
            iv = j * 16 + lanes
            take = (dmin > rm) | ((dmin == rm) & (iv < ri))
            rm = jnp.where(take, dmin, rm)
            ri = jnp.where(take, iv, ri)
            return rm, ri

        rm, ri = upd

        maxv = jnp.max(rm)
        lidx = jnp.min(jnp.where(rm == maxv, ri, jnp.int32(0x7FFFFFFF)))
        p = i & 1
        publish(maxv, (base + lidx).astype(jnp.float32), lidx, p)
        wrow = merge(p)
        outbuf[pl.ds(i * 16, 16)] = wrow
        return wrow[2], wrow[3], wrow[4]

    lax.fori_loop(1, K, step, (wrow0[2], wrow0[3], wrow0[4]))

    def pack(j, _):
        rows = j * 16 + lanes
        x = plsc.load_gather(outbuf, [rows * 16 + 2])
        y = plsc.load_gather(outbuf, [rows * 16 + 3])
        z = plsc.load_gather(outbuf, [rows * 16 + 4])
        plsc.store_scatter(packed, [rows, zeros16], x)
        plsc.store_scatter(packed, [rows, ones16], y)
        plsc.store_scatter(packed, [rows, twos16], z)
        return 0
    lax.fori_loop(0, K // 16, pack, 0)

    @pl.when(is_r0)
    def _():
        pltpu.sync_copy(packed, out_hbm.at[b])


@jax.jit
def _fps(points_list):
    mesh = plsc.VectorSubcoreMesh(core_axis_name="c", subcore_axis_name="s")
    return pl.kernel(
        _fps_body,
        out_type=jax.ShapeDtypeStruct((B, K, 3), jnp.float32),
        mesh=mesh,
        compiler_params=pltpu.CompilerParams(needs_layout_passes=False,
                                             use_tc_tiling_on_sc=False),
        scratch_types=[
            pltpu.VMEM((SHARD, 3), jnp.float32),
            pltpu.VMEM((SHARD,), jnp.float32),
            pltpu.VMEM((SHARD,), jnp.float32),
            pltpu.VMEM((SHARD,), jnp.float32),
            pltpu.VMEM((SHARD,), jnp.float32),
            pltpu.VMEM((16,), jnp.float32),
            pltpu.VMEM((NTILE, 16), jnp.float32),
            pltpu.VMEM((K * 16,), jnp.float32),
            pltpu.VMEM((K, 3), jnp.float32),
            pltpu.VMEM_SHARED((2, 16, 16), jnp.float32),
        ],
    )(points_list)


def kernel(points_list):
    return _fps(points_list)


# speedup vs baseline: 1.0288x; 1.0288x over previous
import jax
import jax.numpy as jnp
from jax import lax
from jax.experimental import pallas as pl
from jax.experimental.pallas import tpu as pltpu
from jax.experimental.pallas import tpu_sc as plsc

B = 4
N = 16384
K = 2048
NTILE = 8
SHARD = N // NTILE
CH = SHARD // 16
UNROLL = 4
BIGF = 3.0e38


def _fps_body(pts_hbm, out_hbm, pts_v, xs, ys, zs, dd, cand_v, candblk,
              outbuf, packed, shared):
    c = lax.axis_index("c")
    s = lax.axis_index("s")
    g = s // NTILE
    r = s % NTILE
    b = c * 2 + g
    base = r * SHARD
    is_r0 = r == 0

    lanes = jnp.arange(16, dtype=jnp.int32)
    zeros16 = jnp.zeros(16, jnp.int32)
    ones16 = jnp.ones(16, jnp.int32)
    twos16 = jnp.full(16, 2, jnp.int32)
    gmask = (lanes // NTILE) == g
    coord_sel = jnp.clip(lanes - 2, 0, 2)
    perms = [lanes ^ k for k in (1, 2, 4, 8)]

    def bfly_argmax(v, i, extra=None):
        for perm in perms:
            vo = jnp.take_along_axis(v, perm, axis=0)
            io = jnp.take_along_axis(i, perm, axis=0)
            t = (vo > v) | ((vo == v) & (io < i))
            v = jnp.where(t, vo, v)
            i = jnp.where(t, io, i)
            if extra is not None:
                eo = jnp.take_along_axis(extra, perm, axis=0)
                extra = jnp.where(t, eo, extra)
        return v, i, extra
    lanes_k = [k * 16 + lanes for k in range(UNROLL)]

    pltpu.sync_copy(pts_hbm.at[b, pl.ds(base, SHARD), :], pts_v)

    def sep(j, _):
        rows = j * 16 + lanes
        sl = pl.ds(j * 16, 16)
        xs[sl] = plsc.load_gather(pts_v, [rows, zeros16])
        ys[sl] = plsc.load_gather(pts_v, [rows, ones16])
        zs[sl] = plsc.load_gather(pts_v, [rows, twos16])
        dd[sl] = jnp.full((16,), 1e10, jnp.float32)
        return 0
    lax.fori_loop(0, CH, sep, 0)

    def publish(val, gif, lidx, p):
        lidxv = zeros16 + lidx
        coords = plsc.load_gather(pts_v, [lidxv, coord_sel])
        cand = jnp.where(lanes == 0, val,
                         jnp.where(lanes == 1, gif, coords))
        cand_v[...] = cand
        pltpu.sync_copy(cand_v, shared.at[p, s])
        plsc.subcore_barrier()

    def merge(p):
        pltpu.sync_copy(shared.at[p, pl.ds(g * NTILE, NTILE)], candblk)
        row_sel = lanes & (NTILE - 1)
        vals = plsc.load_gather(candblk, [row_sel, zeros16])
        idxf = plsc.load_gather(candblk, [row_sel, ones16])
        _, _, rowv = bfly_argmax(vals, idxf, row_sel)
        return plsc.load_gather(candblk, [rowv, lanes])

    publish(jnp.where(is_r0, 1.0, 0.0).astype(jnp.float32),
            jnp.where(is_r0, 0.0, BIGF).astype(jnp.float32),
            jnp.int32(0), 0)
    wrow0 = merge(0)
    outbuf[pl.ds(0, 16)] = wrow0

    def step(i, carry):
        lx, ly, lz = carry

        rm0 = jnp.full((16,), -1.0, jnp.float32)

        @plsc.parallel_loop(0, CH, unroll=UNROLL, carry=(rm0, zeros16))
        def upd(j, mcarry):
            rm, ri = mcarry
            sl = pl.ds(j * 16, 16)
            dx = xs[sl] - lx
            dy = ys[sl] - ly
            dz = zs[sl] - lz
            d = (dx * dx + dy * dy) + dz * dz
            dmin = jnp.minimum(dd[sl], d)
            dd[sl] = dmin
            iv = j * 16 + lanes
            take = (dmin > rm) | ((dmin == rm) & (iv < ri))
            rm = jnp.where(take, dmin, rm)
            ri = jnp.where(take, iv, ri)
            return rm, ri

        rm, ri = upd

        rmw, riw, _ = bfly_argmax(rm, ri)
        maxv = rmw[0]
        lidx = riw[0]
        p = i & 1
        publish(maxv, (base + lidx).astype(jnp.float32), lidx, p)
        wrow = merge(p)
        outbuf[pl.ds(i * 16, 16)] = wrow
        return wrow[2], wrow[3], wrow[4]

    lax.fori_loop(1, K, step, (wrow0[2], wrow0[3], wrow0[4]))

    def pack(j, _):
        rows = j * 16 + lanes
        x = plsc.load_gather(outbuf, [rows * 16 + 2])
        y = plsc.load_gather(outbuf, [rows * 16 + 3])
        z = plsc.load_gather(outbuf, [rows * 16 + 4])
        plsc.store_scatter(packed, [rows, zeros16], x)
        plsc.store_scatter(packed, [rows, ones16], y)
        plsc.store_scatter(packed, [rows, twos16], z)
        return 0
    lax.fori_loop(0, K // 16, pack, 0)

    @pl.when(is_r0)
    def _():
        pltpu.sync_copy(packed, out_hbm.at[b])


@jax.jit
def _fps(points_list):
    mesh = plsc.VectorSubcoreMesh(core_axis_name="c", subcore_axis_name="s")
    return pl.kernel(
        _fps_body,
        out_type=jax.ShapeDtypeStruct((B, K, 3), jnp.float32),
        mesh=mesh,
        compiler_params=pltpu.CompilerParams(needs_layout_passes=False,
                                             use_tc_tiling_on_sc=False),
        scratch_types=[
            pltpu.VMEM((SHARD, 3), jnp.float32),
            pltpu.VMEM((SHARD,), jnp.float32),
            pltpu.VMEM((SHARD,), jnp.float32),
            pltpu.VMEM((SHARD,), jnp.float32),
            pltpu.VMEM((SHARD,), jnp.float32),
            pltpu.VMEM((16,), jnp.float32),
            pltpu.VMEM((NTILE, 16), jnp.float32),
            pltpu.VMEM((K * 16,), jnp.float32),
            pltpu.VMEM((K, 3), jnp.float32),
            pltpu.VMEM_SHARED((2, 16, 16), jnp.float32),
        ],
    )(points_list)


def kernel(points_list):
    return _fps(points_list)
